# baseline (speedup 1.0000x reference)
import functools

import jax
import jax.numpy as jnp
from jax import lax
from jax.experimental import pallas as pl
from jax.experimental.pallas import tpu as pltpu
from jax.experimental.pallas import tpu_sc as plsc

NUM_EMB = 1_000_000
D = 64
B = 4096 * 200
NW = 32
BPW = B // NW
C = 128
NCHUNK = BPW // C

_mesh = plsc.VectorSubcoreMesh(core_axis_name="c", subcore_axis_name="s")


@functools.partial(
    pl.kernel,
    mesh=_mesh,
    compiler_params=pltpu.CompilerParams(use_tc_tiling_on_sc=False),
    out_type=jax.ShapeDtypeStruct((B, D), jnp.float32),
    scratch_types=[
        pltpu.VMEM((NCHUNK, C), jnp.int32),
        pltpu.VMEM((NCHUNK, C), jnp.int32),
        pltpu.VMEM((C, D), jnp.float32),
        pltpu.VMEM((C, D), jnp.float32),
        pltpu.SemaphoreType.DMA,
        pltpu.SemaphoreType.DMA,
    ],
)
def _emb_add(x_hbm, c_hbm, w_hbm, out_hbm, ix_v, ic_v, rx_v, rc_v, s1, s2):
    wid = lax.axis_index("s") * 2 + lax.axis_index("c")
    base = wid * BPW
    pltpu.sync_copy(x_hbm.at[wid], ix_v)
    pltpu.sync_copy(c_hbm.at[wid], ic_v)

    def chunk(i, carry):
        cp1 = pltpu.async_copy(w_hbm.at[ix_v.at[i]], rx_v, s1)
        cp2 = pltpu.async_copy(w_hbm.at[ic_v.at[i]], rc_v, s2)
        cp1.wait()
        cp2.wait()

        def row(r, carry2):
            for j in range(D // 16):
                sl = pl.ds(j * 16, 16)
                rx_v[r, sl] = rx_v[r, sl] + rc_v[r, sl]
            return carry2

        lax.fori_loop(0, C, row, 0, unroll=False)
        pltpu.sync_copy(rx_v, out_hbm.at[pl.ds(base + i * C, C)])
        return carry

    lax.fori_loop(0, NCHUNK, chunk, 0, unroll=False)


def kernel(x, constant, weight):
    x32 = x.astype(jnp.int32).reshape(NW, NCHUNK, C)
    c32 = constant.astype(jnp.int32).reshape(NW, NCHUNK, C)
    out = _emb_add(x32, c32, weight)
    return out.reshape(4096, 200, D)
